# baseline (speedup 1.0000x reference)
import functools

import jax
import jax.numpy as jnp
from jax import lax
from jax.experimental import pallas as pl
from jax.experimental.pallas import tpu as pltpu
from jax.experimental.pallas import tpu_sc as plsc

N = 10000
E = 320000
D_IN = 128
D_H = 64
EPS = 1e-5

NC = 2
NS = 16
NW = NC * NS

BATCH = 128
K = 80
EPW = K * BATCH
E_PAD = NW * EPW
NPAD = 10112
RPT = NPAD // NS
DEG_W = 16

_mesh = plsc.VectorSubcoreMesh(core_axis_name="c", subcore_axis_name="s")
_sc_params = pltpu.CompilerParams(use_tc_tiling_on_sc=False)


@functools.partial(
    pl.kernel,
    mesh=_mesh,
    compiler_params=_sc_params,
    out_type=jax.ShapeDtypeStruct((NC, NPAD, DEG_W), jnp.float32),
    scratch_types=[
        pltpu.VMEM((K, BATCH), jnp.int32),
        pltpu.VMEM((BATCH, DEG_W), jnp.float32),
        pltpu.VMEM_SHARED((NPAD, DEG_W), jnp.float32),
        pltpu.SemaphoreType.DMA,
    ],
)
def _sc_degree(dst_hbm, ones_hbm, zeros_hbm, deg_hbm, dst_v, ones_v, acc_sh, sem):
    c = lax.axis_index("c")
    s = lax.axis_index("s")
    wid = s * NC + c
    pltpu.async_copy(dst_hbm.at[wid], dst_v, sem)
    pltpu.async_copy(ones_hbm, ones_v, sem)
    pltpu.sync_copy(zeros_hbm.at[pl.ds(s * RPT, RPT)], acc_sh.at[pl.ds(s * RPT, RPT)])
    pltpu.make_async_copy(dst_hbm.at[wid], dst_v, sem).wait()
    pltpu.make_async_copy(ones_hbm, ones_v, sem).wait()
    plsc.subcore_barrier()

    def fire(j, carry):
        pltpu.async_copy(ones_v, acc_sh.at[dst_v.at[j]], sem, add=True)
        return carry

    def drain(j, carry):
        pltpu.make_async_copy(ones_v, acc_sh.at[dst_v.at[j]], sem).wait()
        return carry

    lax.fori_loop(0, K, fire, 0)
    lax.fori_loop(0, K, drain, 0)
    plsc.subcore_barrier()
    pltpu.sync_copy(acc_sh.at[pl.ds(s * RPT, RPT)], deg_hbm.at[c, pl.ds(s * RPT, RPT)])


@functools.partial(
    pl.kernel,
    mesh=_mesh,
    compiler_params=_sc_params,
    out_type=jax.ShapeDtypeStruct((NC, NPAD, D_H), jnp.float32),
    scratch_types=[
        pltpu.VMEM((K + 1, BATCH), jnp.int32),
        pltpu.VMEM((K, BATCH), jnp.int32),
        pltpu.VMEM((1, BATCH), jnp.int32),
        pltpu.VMEM((BATCH, D_H), jnp.float32),
        pltpu.VMEM((BATCH, D_H), jnp.float32),
        pltpu.VMEM_SHARED((N, D_H), jnp.float32),
        pltpu.VMEM_SHARED((NPAD, D_H), jnp.float32),
        pltpu.SemaphoreType.DMA,
        pltpu.SemaphoreType.DMA,
        pltpu.SemaphoreType.DMA,
        pltpu.SemaphoreType.DMA,
    ],
)
def _sc_scatter(h_hbm, src_hbm, dst_hbm, pad_hbm, zeros_hbm, out_hbm,
                src_v, dst_v, pad_v, rows_a, rows_b, table_sh, acc_sh,
                ga, gb, sa, sb):
    c = lax.axis_index("c")
    s = lax.axis_index("s")
    wid = s * NC + c
    srt = N // NS
    pltpu.async_copy(src_hbm.at[wid], src_v, ga)
    pltpu.async_copy(dst_hbm.at[wid], dst_v, gb)
    pltpu.async_copy(pad_hbm, pad_v, sa)
    pltpu.async_copy(h_hbm.at[pl.ds(s * srt, srt)], table_sh.at[pl.ds(s * srt, srt)], sb)
    pltpu.sync_copy(zeros_hbm.at[pl.ds(s * RPT, RPT)], acc_sh.at[pl.ds(s * RPT, RPT)])
    pltpu.make_async_copy(src_hbm.at[wid], src_v, ga).wait()
    pltpu.make_async_copy(dst_hbm.at[wid], dst_v, gb).wait()
    pltpu.make_async_copy(pad_hbm, pad_v, sa).wait()
    pltpu.make_async_copy(h_hbm.at[pl.ds(s * srt, srt)], table_sh.at[pl.ds(s * srt, srt)], sb).wait()
    plsc.subcore_barrier()

    pltpu.async_copy(table_sh.at[src_v.at[0]], rows_a, ga)
    pltpu.async_copy(rows_b, acc_sh.at[pad_v.at[0]], sb, add=True)

    def pair(i, carry):
        j = 2 * i
        pltpu.make_async_copy(rows_b, acc_sh.at[pad_v.at[0]], sb).wait()
        pltpu.make_async_copy(table_sh.at[src_v.at[j]], rows_a, ga).wait()
        pltpu.async_copy(table_sh.at[src_v.at[j + 1]], rows_b, gb)
        pltpu.async_copy(rows_a, acc_sh.at[dst_v.at[j]], sa, add=True)
        pltpu.make_async_copy(rows_a, acc_sh.at[pad_v.at[0]], sa).wait()
        pltpu.make_async_copy(table_sh.at[src_v.at[j + 1]], rows_b, gb).wait()
        pltpu.async_copy(table_sh.at[src_v.at[j + 2]], rows_a, ga)
        pltpu.async_copy(rows_b, acc_sh.at[dst_v.at[j + 1]], sb, add=True)
        return carry

    lax.fori_loop(0, K // 2, pair, 0)
    pltpu.make_async_copy(rows_b, acc_sh.at[pad_v.at[0]], sb).wait()
    pltpu.make_async_copy(table_sh.at[src_v.at[K]], rows_a, ga).wait()
    plsc.subcore_barrier()
    pltpu.sync_copy(acc_sh.at[pl.ds(s * RPT, RPT)], out_hbm.at[c, pl.ds(s * RPT, RPT)])


def _tc_mm(x_ref, w0_ref, hsr_ref):
    hsr_ref[...] = jnp.dot(x_ref[...], w0_ref[...],
                           preferred_element_type=jnp.float32)


def _tc_scale(deg_ref, hsr_ref, hs_ref, dinv_ref):
    deg = deg_ref[0, :N, 0:1] + deg_ref[1, :N, 0:1] + 1.0
    dinv = lax.rsqrt(deg)
    dinv_ref[...] = dinv
    hs_ref[...] = hsr_ref[...] * dinv


def _bn_relu(pre, gamma, beta):
    mean = jnp.mean(pre, axis=0, keepdims=True)
    var = jnp.mean((pre - mean) ** 2, axis=0, keepdims=True)
    y = (pre - mean) * lax.rsqrt(var + EPS) * gamma + beta
    return jnp.maximum(y, 0.0)


def _tc_mid(s_ref, hs_ref, dinv_ref, b0_ref, gamma_ref, beta_ref, w1_ref, hs1_ref):
    pre = (s_ref[0, :N, :] + s_ref[1, :N, :] + hs_ref[...]) * dinv_ref[...] + b0_ref[...]
    h = _bn_relu(pre, gamma_ref[...], beta_ref[...])
    hs1_ref[...] = jnp.dot(h * dinv_ref[...], w1_ref[...],
                           preferred_element_type=jnp.float32)


def _tc_post(s_ref, hs_ref, dinv_ref, b1_ref, gamma_ref, beta_ref, out_ref):
    pre = (s_ref[0, :N, :] + s_ref[1, :N, :] + hs_ref[...]) * dinv_ref[...] + b1_ref[...]
    out_ref[...] = _bn_relu(pre, gamma_ref[...], beta_ref[...])


@jax.jit
def kernel(x, edge_index, W0, b0, W1, b1, gamma, beta):
    pad = E_PAD - E
    src = jnp.concatenate([edge_index[0], jnp.zeros((pad,), jnp.int32)])
    dst = jnp.concatenate([edge_index[1], jnp.full((pad,), N, jnp.int32)])
    src3 = src.reshape(NW, K, BATCH)
    dst3 = dst.reshape(NW, K, BATCH)
    src4 = jnp.concatenate([src3, jnp.zeros((NW, 1, BATCH), jnp.int32)], axis=1)
    pad_idx = jnp.full((1, BATCH), N, jnp.int32)

    ones_deg = jnp.ones((BATCH, DEG_W), jnp.float32)
    zeros_deg = jnp.zeros((NPAD, DEG_W), jnp.float32)
    zeros_acc = jnp.zeros((NPAD, D_H), jnp.float32)

    deg_parts = _sc_degree(dst3, ones_deg, zeros_deg)
    hsr = pl.pallas_call(
        _tc_mm,
        out_shape=jax.ShapeDtypeStruct((N, D_H), jnp.float32),
    )(x, W0)

    b0r = b0.reshape(1, D_H)
    b1r = b1.reshape(1, D_H)
    gr = gamma.reshape(1, D_H)
    br = beta.reshape(1, D_H)

    hs0, dinv = pl.pallas_call(
        _tc_scale,
        out_shape=[
            jax.ShapeDtypeStruct((N, D_H), jnp.float32),
            jax.ShapeDtypeStruct((N, 1), jnp.float32),
        ],
    )(deg_parts, hsr)

    s0_parts = _sc_scatter(hs0, src4, dst3, pad_idx, zeros_acc)

    hs1 = pl.pallas_call(
        _tc_mid,
        out_shape=jax.ShapeDtypeStruct((N, D_H), jnp.float32),
    )(s0_parts, hs0, dinv, b0r, gr, br, W1)

    s1_parts = _sc_scatter(hs1, src4, dst3, pad_idx, zeros_acc)

    out = pl.pallas_call(
        _tc_post,
        out_shape=jax.ShapeDtypeStruct((N, D_H), jnp.float32),
    )(s1_parts, hs1, dinv, b1r, gr, br)

    return (out, out)


# speedup vs baseline: 1.0009x; 1.0009x over previous
import functools

import jax
import jax.numpy as jnp
from jax import lax
from jax.experimental import pallas as pl
from jax.experimental.pallas import tpu as pltpu
from jax.experimental.pallas import tpu_sc as plsc

N = 10000
E = 320000
D_IN = 128
D_H = 64
EPS = 1e-5

NC = 2
NS = 16
NW = NC * NS

BATCH = 128
K = 81
EPW = K * BATCH
E_PAD = NW * EPW
NPAD = 10112
RPT = NPAD // NS
DEG_W = 16

_mesh = plsc.VectorSubcoreMesh(core_axis_name="c", subcore_axis_name="s")
_sc_params = pltpu.CompilerParams(use_tc_tiling_on_sc=False)


@functools.partial(
    pl.kernel,
    mesh=_mesh,
    compiler_params=_sc_params,
    out_type=jax.ShapeDtypeStruct((NC, NPAD, DEG_W), jnp.float32),
    scratch_types=[
        pltpu.VMEM((K, BATCH), jnp.int32),
        pltpu.VMEM((BATCH, DEG_W), jnp.float32),
        pltpu.VMEM_SHARED((NPAD, DEG_W), jnp.float32),
        pltpu.SemaphoreType.DMA,
    ],
)
def _sc_degree(dst_hbm, ones_hbm, zeros_hbm, deg_hbm, dst_v, ones_v, acc_sh, sem):
    c = lax.axis_index("c")
    s = lax.axis_index("s")
    wid = s * NC + c
    pltpu.async_copy(dst_hbm.at[wid], dst_v, sem)
    pltpu.async_copy(ones_hbm, ones_v, sem)
    pltpu.sync_copy(zeros_hbm.at[pl.ds(s * RPT, RPT)], acc_sh.at[pl.ds(s * RPT, RPT)])
    pltpu.make_async_copy(dst_hbm.at[wid], dst_v, sem).wait()
    pltpu.make_async_copy(ones_hbm, ones_v, sem).wait()
    plsc.subcore_barrier()

    def fire(j, carry):
        pltpu.async_copy(ones_v, acc_sh.at[dst_v.at[j]], sem, add=True)
        return carry

    def drain(j, carry):
        pltpu.make_async_copy(ones_v, acc_sh.at[dst_v.at[j]], sem).wait()
        return carry

    lax.fori_loop(0, K, fire, 0)
    lax.fori_loop(0, K, drain, 0)
    plsc.subcore_barrier()
    pltpu.sync_copy(acc_sh.at[pl.ds(s * RPT, RPT)], deg_hbm.at[c, pl.ds(s * RPT, RPT)])


@functools.partial(
    pl.kernel,
    mesh=_mesh,
    compiler_params=_sc_params,
    out_type=jax.ShapeDtypeStruct((NC, NPAD, D_H), jnp.float32),
    scratch_types=[
        pltpu.VMEM((K + 2, BATCH), jnp.int32),
        pltpu.VMEM((K, BATCH), jnp.int32),
        pltpu.VMEM((1, BATCH), jnp.int32),
        pltpu.VMEM((BATCH, D_H), jnp.float32),
        pltpu.VMEM((BATCH, D_H), jnp.float32),
        pltpu.VMEM((BATCH, D_H), jnp.float32),
        pltpu.VMEM_SHARED((N, D_H), jnp.float32),
        pltpu.VMEM_SHARED((NPAD, D_H), jnp.float32),
        pltpu.SemaphoreType.DMA,
        pltpu.SemaphoreType.DMA,
        pltpu.SemaphoreType.DMA,
        pltpu.SemaphoreType.DMA,
        pltpu.SemaphoreType.DMA,
        pltpu.SemaphoreType.DMA,
    ],
)
def _sc_scatter(h_hbm, src_hbm, dst_hbm, pad_hbm, zeros_hbm, out_hbm,
                src_v, dst_v, pad_v, r0, r1, r2, table_sh, acc_sh,
                g0, g1, g2, s0, s1, s2):
    c = lax.axis_index("c")
    s = lax.axis_index("s")
    wid = s * NC + c
    rows = (r0, r1, r2)
    gs = (g0, g1, g2)
    ss = (s0, s1, s2)
    ga, gb, sa, sb = g0, g1, s0, s1
    srt = N // NS
    pltpu.async_copy(src_hbm.at[wid], src_v, ga)
    pltpu.async_copy(dst_hbm.at[wid], dst_v, gb)
    pltpu.async_copy(pad_hbm, pad_v, sa)
    pltpu.async_copy(h_hbm.at[pl.ds(s * srt, srt)], table_sh.at[pl.ds(s * srt, srt)], sb)
    pltpu.sync_copy(zeros_hbm.at[pl.ds(s * RPT, RPT)], acc_sh.at[pl.ds(s * RPT, RPT)])
    pltpu.make_async_copy(src_hbm.at[wid], src_v, ga).wait()
    pltpu.make_async_copy(dst_hbm.at[wid], dst_v, gb).wait()
    pltpu.make_async_copy(pad_hbm, pad_v, sa).wait()
    pltpu.make_async_copy(h_hbm.at[pl.ds(s * srt, srt)], table_sh.at[pl.ds(s * srt, srt)], sb).wait()
    plsc.subcore_barrier()

    pltpu.async_copy(table_sh.at[src_v.at[0]], rows[0], gs[0])
    pltpu.async_copy(table_sh.at[src_v.at[1]], rows[1], gs[1])
    pltpu.async_copy(rows[2], acc_sh.at[pad_v.at[0]], ss[2], add=True)

    def body(i, carry):
        j = 3 * i
        for o in range(3):
            b = o
            bp = (o + 2) % 3
            pltpu.make_async_copy(rows[bp], acc_sh.at[pad_v.at[0]], ss[bp]).wait()
            pltpu.async_copy(table_sh.at[src_v.at[j + o + 2]], rows[bp], gs[bp])
            pltpu.make_async_copy(table_sh.at[src_v.at[j + o]], rows[b], gs[b]).wait()
            pltpu.async_copy(rows[b], acc_sh.at[dst_v.at[j + o]], ss[b], add=True)
        return carry

    lax.fori_loop(0, K // 3, body, 0)
    pltpu.make_async_copy(table_sh.at[src_v.at[K]], rows[K % 3], gs[K % 3]).wait()
    pltpu.make_async_copy(table_sh.at[src_v.at[K + 1]], rows[(K + 1) % 3], gs[(K + 1) % 3]).wait()
    pltpu.make_async_copy(rows[(K - 1) % 3], acc_sh.at[pad_v.at[0]], ss[(K - 1) % 3]).wait()
    plsc.subcore_barrier()
    pltpu.sync_copy(acc_sh.at[pl.ds(s * RPT, RPT)], out_hbm.at[c, pl.ds(s * RPT, RPT)])


def _tc_pre(deg_ref, x_ref, w0_ref, hs_ref, dinv_ref):
    deg = deg_ref[0, :N, 0:1] + deg_ref[1, :N, 0:1] + 1.0
    dinv = lax.rsqrt(deg)
    dinv_ref[...] = dinv
    hs_ref[...] = jnp.dot(x_ref[...] * dinv, w0_ref[...],
                          preferred_element_type=jnp.float32)


def _bn_relu(pre, gamma, beta):
    mean = jnp.mean(pre, axis=0, keepdims=True)
    var = jnp.mean((pre - mean) ** 2, axis=0, keepdims=True)
    y = (pre - mean) * lax.rsqrt(var + EPS) * gamma + beta
    return jnp.maximum(y, 0.0)


def _tc_mid(s_ref, hs_ref, dinv_ref, b0_ref, gamma_ref, beta_ref, w1_ref, hs1_ref):
    pre = (s_ref[0, :N, :] + s_ref[1, :N, :] + hs_ref[...]) * dinv_ref[...] + b0_ref[...]
    h = _bn_relu(pre, gamma_ref[...], beta_ref[...])
    hs1_ref[...] = jnp.dot(h * dinv_ref[...], w1_ref[...],
                           preferred_element_type=jnp.float32)


def _tc_post(s_ref, hs_ref, dinv_ref, b1_ref, gamma_ref, beta_ref, out_ref):
    pre = (s_ref[0, :N, :] + s_ref[1, :N, :] + hs_ref[...]) * dinv_ref[...] + b1_ref[...]
    out_ref[...] = _bn_relu(pre, gamma_ref[...], beta_ref[...])


@jax.jit
def kernel(x, edge_index, W0, b0, W1, b1, gamma, beta):
    pad = E_PAD - E
    src = jnp.concatenate([edge_index[0], jnp.zeros((pad,), jnp.int32)])
    dst = jnp.concatenate([edge_index[1], jnp.full((pad,), N, jnp.int32)])
    src3 = src.reshape(NW, K, BATCH)
    dst3 = dst.reshape(NW, K, BATCH)
    src4 = jnp.concatenate([src3, jnp.zeros((NW, 2, BATCH), jnp.int32)], axis=1)
    pad_idx = jnp.full((1, BATCH), N, jnp.int32)

    ones_deg = jnp.ones((BATCH, DEG_W), jnp.float32)
    zeros_deg = jnp.zeros((NPAD, DEG_W), jnp.float32)
    zeros_acc = jnp.zeros((NPAD, D_H), jnp.float32)

    deg_parts = _sc_degree(dst3, ones_deg, zeros_deg)

    b0r = b0.reshape(1, D_H)
    b1r = b1.reshape(1, D_H)
    gr = gamma.reshape(1, D_H)
    br = beta.reshape(1, D_H)

    hs0, dinv = pl.pallas_call(
        _tc_pre,
        out_shape=[
            jax.ShapeDtypeStruct((N, D_H), jnp.float32),
            jax.ShapeDtypeStruct((N, 1), jnp.float32),
        ],
    )(deg_parts, x, W0)

    s0_parts = _sc_scatter(hs0, src4, dst3, pad_idx, zeros_acc)

    hs1 = pl.pallas_call(
        _tc_mid,
        out_shape=jax.ShapeDtypeStruct((N, D_H), jnp.float32),
    )(s0_parts, hs0, dinv, b0r, gr, br, W1)

    s1_parts = _sc_scatter(hs1, src4, dst3, pad_idx, zeros_acc)

    out = pl.pallas_call(
        _tc_post,
        out_shape=jax.ShapeDtypeStruct((N, D_H), jnp.float32),
    )(s1_parts, hs1, dinv, b1r, gr, br)

    return (out, out)


# speedup vs baseline: 1.0054x; 1.0045x over previous
import functools

import jax
import jax.numpy as jnp
from jax import lax
from jax.experimental import pallas as pl
from jax.experimental.pallas import tpu as pltpu
from jax.experimental.pallas import tpu_sc as plsc

N = 10000
E = 320000
D_IN = 128
D_H = 64
EPS = 1e-5

NC = 2
NS = 16
NW = NC * NS

BATCH = 128
K = 80
EPW = K * BATCH
E_PAD = NW * EPW
NPAD = 10112
RPT = NPAD // NS
DEG_W = 16

_mesh = plsc.VectorSubcoreMesh(core_axis_name="c", subcore_axis_name="s")
_sc_params = pltpu.CompilerParams(use_tc_tiling_on_sc=False)


@functools.partial(
    pl.kernel,
    mesh=_mesh,
    compiler_params=_sc_params,
    out_type=jax.ShapeDtypeStruct((NC, NPAD, DEG_W), jnp.float32),
    scratch_types=[
        pltpu.VMEM((K, BATCH), jnp.int32),
        pltpu.VMEM((BATCH, DEG_W), jnp.float32),
        pltpu.VMEM_SHARED((NPAD, DEG_W), jnp.float32),
        pltpu.SemaphoreType.DMA,
    ],
)
def _sc_degree(dst_hbm, ones_hbm, zeros_hbm, deg_hbm, dst_v, ones_v, acc_sh, sem):
    c = lax.axis_index("c")
    s = lax.axis_index("s")
    wid = s * NC + c
    pltpu.async_copy(dst_hbm.at[wid], dst_v, sem)
    pltpu.async_copy(ones_hbm, ones_v, sem)
    pltpu.sync_copy(zeros_hbm.at[pl.ds(s * RPT, RPT)], acc_sh.at[pl.ds(s * RPT, RPT)])
    pltpu.make_async_copy(dst_hbm.at[wid], dst_v, sem).wait()
    pltpu.make_async_copy(ones_hbm, ones_v, sem).wait()
    plsc.subcore_barrier()

    def fire(j, carry):
        pltpu.async_copy(ones_v, acc_sh.at[dst_v.at[j]], sem, add=True)
        return carry

    def drain(j, carry):
        pltpu.make_async_copy(ones_v, acc_sh.at[dst_v.at[j]], sem).wait()
        return carry

    lax.fori_loop(0, K, fire, 0)
    lax.fori_loop(0, K, drain, 0)
    plsc.subcore_barrier()
    pltpu.sync_copy(acc_sh.at[pl.ds(s * RPT, RPT)], deg_hbm.at[c, pl.ds(s * RPT, RPT)])


@functools.partial(
    pl.kernel,
    mesh=_mesh,
    compiler_params=_sc_params,
    out_type=jax.ShapeDtypeStruct((NC, NPAD, D_H), jnp.float32),
    scratch_types=[
        pltpu.VMEM((K + 1, BATCH), jnp.int32),
        pltpu.VMEM((K, BATCH), jnp.int32),
        pltpu.VMEM((1, BATCH), jnp.int32),
        pltpu.VMEM((BATCH, D_H), jnp.float32),
        pltpu.VMEM((BATCH, D_H), jnp.float32),
        pltpu.VMEM_SHARED((N, D_H), jnp.float32),
        pltpu.VMEM_SHARED((NPAD, D_H), jnp.float32),
        pltpu.SemaphoreType.DMA,
        pltpu.SemaphoreType.DMA,
        pltpu.SemaphoreType.DMA,
        pltpu.SemaphoreType.DMA,
    ],
)
def _sc_scatter(h_hbm, src_hbm, dst_hbm, pad_hbm, zeros_hbm, out_hbm,
                src_v, dst_v, pad_v, rows_a, rows_b, table_sh, acc_sh,
                ga, gb, sa, sb):
    c = lax.axis_index("c")
    s = lax.axis_index("s")
    wid = s * NC + c
    srt = N // NS
    pltpu.async_copy(src_hbm.at[wid], src_v, ga)
    pltpu.async_copy(dst_hbm.at[wid], dst_v, gb)
    pltpu.async_copy(pad_hbm, pad_v, sa)
    pltpu.async_copy(h_hbm.at[pl.ds(s * srt, srt)], table_sh.at[pl.ds(s * srt, srt)], sb)
    pltpu.sync_copy(zeros_hbm.at[pl.ds(s * RPT, RPT)], acc_sh.at[pl.ds(s * RPT, RPT)])
    pltpu.make_async_copy(src_hbm.at[wid], src_v, ga).wait()
    pltpu.make_async_copy(dst_hbm.at[wid], dst_v, gb).wait()
    pltpu.make_async_copy(pad_hbm, pad_v, sa).wait()
    pltpu.make_async_copy(h_hbm.at[pl.ds(s * srt, srt)], table_sh.at[pl.ds(s * srt, srt)], sb).wait()
    plsc.subcore_barrier()

    pltpu.async_copy(table_sh.at[src_v.at[0]], rows_a, ga)
    pltpu.async_copy(rows_b, acc_sh.at[pad_v.at[0]], sb, add=True)

    def pair(i, carry):
        j = 2 * i
        pltpu.make_async_copy(rows_b, acc_sh.at[pad_v.at[0]], sb).wait()
        pltpu.make_async_copy(table_sh.at[src_v.at[j]], rows_a, ga).wait()
        pltpu.async_copy(table_sh.at[src_v.at[j + 1]], rows_b, gb)
        pltpu.async_copy(rows_a, acc_sh.at[dst_v.at[j]], sa, add=True)
        pltpu.make_async_copy(rows_a, acc_sh.at[pad_v.at[0]], sa).wait()
        pltpu.make_async_copy(table_sh.at[src_v.at[j + 1]], rows_b, gb).wait()
        pltpu.async_copy(table_sh.at[src_v.at[j + 2]], rows_a, ga)
        pltpu.async_copy(rows_b, acc_sh.at[dst_v.at[j + 1]], sb, add=True)
        return carry

    lax.fori_loop(0, K // 2, pair, 0)
    pltpu.make_async_copy(rows_b, acc_sh.at[pad_v.at[0]], sb).wait()
    pltpu.make_async_copy(table_sh.at[src_v.at[K]], rows_a, ga).wait()
    plsc.subcore_barrier()
    pltpu.sync_copy(acc_sh.at[pl.ds(s * RPT, RPT)], out_hbm.at[c, pl.ds(s * RPT, RPT)])


def _tc_pre(deg_ref, x_ref, w0_ref, hs_ref, dinv_ref):
    deg = deg_ref[0, :N, 0:1] + deg_ref[1, :N, 0:1] + 1.0
    dinv = lax.rsqrt(deg)
    dinv_ref[...] = dinv
    hs_ref[...] = jnp.dot(x_ref[...] * dinv, w0_ref[...],
                          preferred_element_type=jnp.float32)


def _bn_relu(pre, gamma, beta):
    mean = jnp.mean(pre, axis=0, keepdims=True)
    var = jnp.mean((pre - mean) ** 2, axis=0, keepdims=True)
    y = (pre - mean) * lax.rsqrt(var + EPS) * gamma + beta
    return jnp.maximum(y, 0.0)


def _tc_mid(s_ref, hs_ref, dinv_ref, b0_ref, gamma_ref, beta_ref, w1_ref, hs1_ref):
    pre = (s_ref[0, :N, :] + s_ref[1, :N, :] + hs_ref[...]) * dinv_ref[...] + b0_ref[...]
    h = _bn_relu(pre, gamma_ref[...], beta_ref[...])
    hs1_ref[...] = jnp.dot(h * dinv_ref[...], w1_ref[...],
                           preferred_element_type=jnp.float32)


def _tc_post(s_ref, hs_ref, dinv_ref, b1_ref, gamma_ref, beta_ref, out_ref):
    pre = (s_ref[0, :N, :] + s_ref[1, :N, :] + hs_ref[...]) * dinv_ref[...] + b1_ref[...]
    out_ref[...] = _bn_relu(pre, gamma_ref[...], beta_ref[...])


@jax.jit
def kernel(x, edge_index, W0, b0, W1, b1, gamma, beta):
    pad = E_PAD - E
    src = jnp.concatenate([edge_index[0], jnp.zeros((pad,), jnp.int32)])
    dst = jnp.concatenate([edge_index[1], jnp.full((pad,), N, jnp.int32)])
    src3 = src.reshape(NW, K, BATCH)
    dst3 = dst.reshape(NW, K, BATCH)
    src4 = jnp.concatenate([src3, jnp.zeros((NW, 1, BATCH), jnp.int32)], axis=1)
    pad_idx = jnp.full((1, BATCH), N, jnp.int32)

    ones_deg = jnp.ones((BATCH, DEG_W), jnp.float32)
    zeros_deg = jnp.zeros((NPAD, DEG_W), jnp.float32)
    zeros_acc = jnp.zeros((NPAD, D_H), jnp.float32)

    deg_parts = _sc_degree(dst3, ones_deg, zeros_deg)

    b0r = b0.reshape(1, D_H)
    b1r = b1.reshape(1, D_H)
    gr = gamma.reshape(1, D_H)
    br = beta.reshape(1, D_H)

    hs0, dinv = pl.pallas_call(
        _tc_pre,
        out_shape=[
            jax.ShapeDtypeStruct((N, D_H), jnp.float32),
            jax.ShapeDtypeStruct((N, 1), jnp.float32),
        ],
    )(deg_parts, x, W0)

    s0_parts = _sc_scatter(hs0, src4, dst3, pad_idx, zeros_acc)

    hs1 = pl.pallas_call(
        _tc_mid,
        out_shape=jax.ShapeDtypeStruct((N, D_H), jnp.float32),
    )(s0_parts, hs0, dinv, b0r, gr, br, W1)

    s1_parts = _sc_scatter(hs1, src4, dst3, pad_idx, zeros_acc)

    out = pl.pallas_call(
        _tc_post,
        out_shape=jax.ShapeDtypeStruct((N, D_H), jnp.float32),
    )(s1_parts, hs1, dinv, b1r, gr, br)

    return (out, out)
